# baseline (speedup 1.0000x reference)
import jax
import jax.numpy as jnp
from jax.experimental import pallas as pl
from jax.experimental.pallas import tpu as pltpu

B = 16
V = 16
D = 512
H = 8
HD = D // H
L = 2
F = 1376
S = 512
K = 512
POOL = (512, 256, 128)
NC = 128
REP = S // NC

_DN_T = (((1,), (1,)), ((), ()))


def _q(x):
    return x.astype(jnp.bfloat16).astype(jnp.float32)


def _dot(a, b):
    return jnp.dot(a, b, preferred_element_type=jnp.float32)


def _dot_t(a, b):
    return jax.lax.dot_general(a, b, _DN_T,
                               preferred_element_type=jnp.float32)


def _hdot(a, b):
    return jnp.dot(a, b, precision=jax.lax.Precision.HIGHEST,
                   preferred_element_type=jnp.float32)


def _rms(x, s):
    return (x * jax.lax.rsqrt(jnp.mean(x * x, axis=-1, keepdims=True) + 1e-6)) * s


def _ffn(h, n2, w1, w3, w2):
    xn2 = _rms(h, n2)
    g = _dot(xn2, w1)
    u = _dot(xn2, w3)
    return h + _dot(jax.nn.silu(g) * u, w2)


def _block_body(h, cnt, n1, wq, wk, wv, wo, n2, w1, w3, w2):
    xn = _rms(h, n1)
    q = _dot(xn, wq)
    k = _dot(xn, wk)
    v = _dot(xn, wv)
    scale = 1.0 / (HD ** 0.5)
    outs = []
    for hh in range(H):
        sl = slice(hh * HD, (hh + 1) * HD)
        s = _dot_t(q[:, sl], k[:, sl]) * scale
        e = jnp.exp(s)
        if cnt is None:
            a = e * jax.lax.reciprocal(jnp.sum(e, axis=-1, keepdims=True))
            outs.append(_dot(a, v[:, sl]))
        else:
            a = e * jax.lax.reciprocal(
                jnp.sum(e * cnt, axis=-1, keepdims=True))
            outs.append(_hdot(_q(a) * cnt, _q(v[:, sl])))
    attn = _dot(jnp.concatenate(outs, axis=1), wo)
    return _ffn(h + attn, n2, w1, w3, w2)


def _pool_attn(q0, h, cnt, wk, wv, wo):
    kt = _dot(h, wk)
    vt = _dot(h, wv)
    s = _dot_t(q0, kt) * (D ** -0.5)
    e = jnp.exp(s)
    if cnt is None:
        a = e * jax.lax.reciprocal(jnp.sum(e, axis=-1, keepdims=True))
        av = _dot(a, vt)
    else:
        a = e * jax.lax.reciprocal(jnp.sum(e * cnt, axis=-1, keepdims=True))
        av = _hdot(_q(a) * cnt, _q(vt))
    return _dot(av, wo)


def _enc_kern(x_ref, emb_ref, n1_ref, n2_ref, wq_ref, wk_ref, wv_ref,
              wo_ref, w1_ref, w3_ref, w2_ref,
              pq0_ref, pq1_ref, pq2_ref, pwq_ref, pwk_ref, pwv_ref,
              pwo_ref, cb_ref, zq_ref, idx_ref, q0_scr, q1_scr, q2_scr):
    b = pl.program_id(0)

    @pl.when(b == 0)
    def _():
        q0_scr[...] = _dot(pq0_ref[...], pwq_ref[0])
        q1_scr[...] = _dot(pq1_ref[...], pwq_ref[1])
        q2_scr[...] = _dot(pq2_ref[...], pwq_ref[2])

    xv = x_ref[0]
    iota = jax.lax.broadcasted_iota(jnp.int32, (V, S), 0)
    onehot = (iota == xv).astype(jnp.float32)
    cnt = jnp.transpose(jnp.sum(onehot, axis=-1, keepdims=True))

    h = emb_ref[...]
    for i in range(L):
        h = _block_body(h, cnt, n1_ref[i][None], wq_ref[i], wk_ref[i],
                        wv_ref[i], wo_ref[i], n2_ref[i][None], w1_ref[i],
                        w3_ref[i], w2_ref[i])

    h = _pool_attn(q0_scr[...], h, cnt, pwk_ref[0], pwv_ref[0], pwo_ref[0])
    h = _pool_attn(q1_scr[...], h, None, pwk_ref[1], pwv_ref[1], pwo_ref[1])
    h = _pool_attn(q2_scr[...], h, None, pwk_ref[2], pwv_ref[2], pwo_ref[2])

    cb = cb_ref[...]
    zz = jnp.sum(h * h, axis=-1, keepdims=True)
    cc = jnp.transpose(jnp.sum(cb * cb, axis=-1, keepdims=True))
    zcb = _dot_t(h, cb)
    d2 = zz - 2.0 * zcb + cc
    idx = jnp.argmin(d2, axis=-1)
    sel = (jax.lax.broadcasted_iota(jnp.int32, (NC, K), 1)
           == idx[:, None]).astype(jnp.float32)
    zq_ref[0] = _hdot(sel, cb)
    idx_ref[0] = idx.reshape(1, NC)


def _encoder(x3, emb, n1, n2, wq, wk, wv, wo, w1, w3, w2,
             pq0, pq1, pq2, pwq, pwk, pwv, pwo, cb):
    wspec = pl.BlockSpec((L, D, D), lambda b: (0, 0, 0))
    pspec = pl.BlockSpec((3, D, D), lambda b: (0, 0, 0))
    return pl.pallas_call(
        _enc_kern,
        grid=(B,),
        in_specs=[
            pl.BlockSpec((1, 1, S), lambda b: (b, 0, 0)),
            pl.BlockSpec((V, D), lambda b: (0, 0)),
            pl.BlockSpec((L, D), lambda b: (0, 0)),
            pl.BlockSpec((L, D), lambda b: (0, 0)),
            wspec, wspec, wspec, wspec,
            pl.BlockSpec((L, D, F), lambda b: (0, 0, 0)),
            pl.BlockSpec((L, D, F), lambda b: (0, 0, 0)),
            pl.BlockSpec((L, F, D), lambda b: (0, 0, 0)),
            pl.BlockSpec((POOL[0], D), lambda b: (0, 0)),
            pl.BlockSpec((POOL[1], D), lambda b: (0, 0)),
            pl.BlockSpec((POOL[2], D), lambda b: (0, 0)),
            pspec, pspec, pspec, pspec,
            pl.BlockSpec((K, D), lambda b: (0, 0)),
        ],
        out_specs=[
            pl.BlockSpec((1, NC, D), lambda b: (b, 0, 0)),
            pl.BlockSpec((1, 1, NC), lambda b: (b, 0, 0)),
        ],
        out_shape=[
            jax.ShapeDtypeStruct((B, NC, D), jnp.float32),
            jax.ShapeDtypeStruct((B, 1, NC), jnp.int32),
        ],
        scratch_shapes=[
            pltpu.VMEM((POOL[0], D), jnp.float32),
            pltpu.VMEM((POOL[1], D), jnp.float32),
            pltpu.VMEM((POOL[2], D), jnp.float32),
        ],
    )(x3, emb, n1, n2, wq, wk, wv, wo, w1, w3, w2,
      pq0, pq1, pq2, pwq, pwk, pwv, pwo, cb)


def _dec_kern(h_ref, n1_ref, n2_ref, wq_ref, wk_ref, wv_ref, wo_ref,
              w1_ref, w3_ref, w2_ref, fin_ref, head_ref, o_ref):
    h = h_ref[0]
    for i in range(L):
        h = _block_body(h, None, n1_ref[i][None], wq_ref[i], wk_ref[i],
                        wv_ref[i], wo_ref[i], n2_ref[i][None], w1_ref[i],
                        w3_ref[i], w2_ref[i])
    xn = _rms(h, fin_ref[...])
    lg = _dot(xn, head_ref[...])
    o_ref[0] = jnp.broadcast_to(lg[:, None, :], (NC, REP, V)).reshape(S, V)


def _decoder(h, n1, n2, wq, wk, wv, wo, w1, w3, w2, fin_n, head):
    wspec = pl.BlockSpec((L, D, D), lambda b: (0, 0, 0))
    return pl.pallas_call(
        _dec_kern,
        grid=(B,),
        in_specs=[
            pl.BlockSpec((1, NC, D), lambda b: (b, 0, 0)),
            pl.BlockSpec((L, D), lambda b: (0, 0)),
            pl.BlockSpec((L, D), lambda b: (0, 0)),
            wspec, wspec, wspec, wspec,
            pl.BlockSpec((L, D, F), lambda b: (0, 0, 0)),
            pl.BlockSpec((L, D, F), lambda b: (0, 0, 0)),
            pl.BlockSpec((L, F, D), lambda b: (0, 0, 0)),
            pl.BlockSpec((1, D), lambda b: (0, 0)),
            pl.BlockSpec((D, V), lambda b: (0, 0)),
        ],
        out_specs=pl.BlockSpec((1, S, V), lambda b: (b, 0, 0)),
        out_shape=jax.ShapeDtypeStruct((B, S, V), jnp.float32),
    )(h, n1, n2, wq, wk, wv, wo, w1, w3, w2, fin_n, head)


def kernel(x, tok_emb, enc_wq, enc_wk, enc_wv, enc_wo, enc_w1, enc_w3,
           enc_w2, enc_n1, enc_n2, dec_wq, dec_wk, dec_wv, dec_wo, dec_w1,
           dec_w3, dec_w2, dec_n1, dec_n2, pq0, pq1, pq2, p_wq, p_wk, p_wv,
           p_wo, codebook, fin_n, head):
    zq, idx3 = _encoder(x.reshape(B, 1, S).astype(jnp.int32), tok_emb,
                        enc_n1, enc_n2, enc_wq, enc_wk, enc_wv, enc_wo,
                        enc_w1, enc_w3, enc_w2, pq0, pq1, pq2, p_wq, p_wk,
                        p_wv, p_wo, codebook)
    logits = _decoder(zq, dec_n1, dec_n2, dec_wq, dec_wk, dec_wv, dec_wo,
                      dec_w1, dec_w3, dec_w2, fin_n[None], head)
    return logits, idx3.reshape(B, NC)


# speedup vs baseline: 1.0109x; 1.0109x over previous
import jax
import jax.numpy as jnp
from jax.experimental import pallas as pl
from jax.experimental.pallas import tpu as pltpu

B = 16
V = 16
D = 512
H = 8
HD = D // H
L = 2
F = 1376
S = 512
K = 512
POOL = (512, 256, 128)
NC = 128
REP = S // NC
EG = 2
DG = 4

_DN_T = (((1,), (1,)), ((), ()))


def _q(x):
    return x.astype(jnp.bfloat16).astype(jnp.float32)


def _dot(a, b):
    return jnp.dot(a, b, preferred_element_type=jnp.float32)


def _dot_t(a, b):
    return jax.lax.dot_general(a, b, _DN_T,
                               preferred_element_type=jnp.float32)


def _hdot(a, b):
    return jnp.dot(a, b, precision=jax.lax.Precision.HIGHEST,
                   preferred_element_type=jnp.float32)


def _rms(x, s):
    return (x * jax.lax.rsqrt(jnp.mean(x * x, axis=-1, keepdims=True) + 1e-6)) * s


def _ffn(h, n2, w1, w3, w2):
    xn2 = _rms(h, n2)
    g = _dot(xn2, w1)
    u = _dot(xn2, w3)
    return h + _dot(jax.nn.silu(g) * u, w2)


def _block_body(h, cnt, n1, wq, wk, wv, wo, n2, w1, w3, w2):
    xn = _rms(h, n1)
    q = _dot(xn, wq)
    k = _dot(xn, wk)
    v = _dot(xn, wv)
    scale = 1.0 / (HD ** 0.5)
    outs = []
    for hh in range(H):
        sl = slice(hh * HD, (hh + 1) * HD)
        s = _dot_t(q[:, sl], k[:, sl]) * scale
        e = jnp.exp(s)
        if cnt is None:
            a = e * jax.lax.reciprocal(jnp.sum(e, axis=-1, keepdims=True))
            outs.append(_dot(a, v[:, sl]))
        else:
            a = e * jax.lax.reciprocal(
                jnp.sum(e * cnt, axis=-1, keepdims=True))
            outs.append(_hdot(_q(a) * cnt, _q(v[:, sl])))
    attn = _dot(jnp.concatenate(outs, axis=1), wo)
    return _ffn(h + attn, n2, w1, w3, w2)


def _pool_attn(q0, h, cnt, wk, wv, wo):
    kt = _dot(h, wk)
    vt = _dot(h, wv)
    s = _dot_t(q0, kt) * (D ** -0.5)
    e = jnp.exp(s)
    if cnt is None:
        a = e * jax.lax.reciprocal(jnp.sum(e, axis=-1, keepdims=True))
        av = _dot(a, vt)
    else:
        a = e * jax.lax.reciprocal(jnp.sum(e * cnt, axis=-1, keepdims=True))
        av = _hdot(_q(a) * cnt, _q(vt))
    return _dot(av, wo)


def _enc_kern(x_ref, emb_ref, n1_ref, n2_ref, wq_ref, wk_ref, wv_ref,
              wo_ref, w1_ref, w3_ref, w2_ref,
              pq0_ref, pq1_ref, pq2_ref, pwq_ref, pwk_ref, pwv_ref,
              pwo_ref, cb_ref, zq_ref, idx_ref, q0_scr, q1_scr, q2_scr):
    b = pl.program_id(0)

    @pl.when(b == 0)
    def _():
        q0_scr[...] = _dot(pq0_ref[...], pwq_ref[0])
        q1_scr[...] = _dot(pq1_ref[...], pwq_ref[1])
        q2_scr[...] = _dot(pq2_ref[...], pwq_ref[2])

    cb = cb_ref[...]
    cc = jnp.transpose(jnp.sum(cb * cb, axis=-1, keepdims=True))
    iota = jax.lax.broadcasted_iota(jnp.int32, (V, S), 0)
    for j in range(EG):
        xv = x_ref[j]
        onehot = (iota == xv).astype(jnp.float32)
        cnt = jnp.transpose(jnp.sum(onehot, axis=-1, keepdims=True))

        h = emb_ref[...]
        for i in range(L):
            h = _block_body(h, cnt, n1_ref[i][None], wq_ref[i], wk_ref[i],
                            wv_ref[i], wo_ref[i], n2_ref[i][None], w1_ref[i],
                            w3_ref[i], w2_ref[i])

        h = _pool_attn(q0_scr[...], h, cnt, pwk_ref[0], pwv_ref[0],
                       pwo_ref[0])
        h = _pool_attn(q1_scr[...], h, None, pwk_ref[1], pwv_ref[1],
                       pwo_ref[1])
        h = _pool_attn(q2_scr[...], h, None, pwk_ref[2], pwv_ref[2],
                       pwo_ref[2])

        zz = jnp.sum(h * h, axis=-1, keepdims=True)
        zcb = _dot_t(h, cb)
        d2 = zz - 2.0 * zcb + cc
        idx = jnp.argmin(d2, axis=-1)
        sel = (jax.lax.broadcasted_iota(jnp.int32, (NC, K), 1)
               == idx[:, None]).astype(jnp.float32)
        zq_ref[j] = _hdot(sel, cb)
        idx_ref[j] = idx.reshape(1, NC)


def _encoder(x3, emb, n1, n2, wq, wk, wv, wo, w1, w3, w2,
             pq0, pq1, pq2, pwq, pwk, pwv, pwo, cb):
    wspec = pl.BlockSpec((L, D, D), lambda b: (0, 0, 0))
    pspec = pl.BlockSpec((3, D, D), lambda b: (0, 0, 0))
    return pl.pallas_call(
        _enc_kern,
        grid=(B // EG,),
        in_specs=[
            pl.BlockSpec((EG, 1, S), lambda b: (b, 0, 0)),
            pl.BlockSpec((V, D), lambda b: (0, 0)),
            pl.BlockSpec((L, D), lambda b: (0, 0)),
            pl.BlockSpec((L, D), lambda b: (0, 0)),
            wspec, wspec, wspec, wspec,
            pl.BlockSpec((L, D, F), lambda b: (0, 0, 0)),
            pl.BlockSpec((L, D, F), lambda b: (0, 0, 0)),
            pl.BlockSpec((L, F, D), lambda b: (0, 0, 0)),
            pl.BlockSpec((POOL[0], D), lambda b: (0, 0)),
            pl.BlockSpec((POOL[1], D), lambda b: (0, 0)),
            pl.BlockSpec((POOL[2], D), lambda b: (0, 0)),
            pspec, pspec, pspec, pspec,
            pl.BlockSpec((K, D), lambda b: (0, 0)),
        ],
        out_specs=[
            pl.BlockSpec((EG, NC, D), lambda b: (b, 0, 0)),
            pl.BlockSpec((EG, 1, NC), lambda b: (b, 0, 0)),
        ],
        out_shape=[
            jax.ShapeDtypeStruct((B, NC, D), jnp.float32),
            jax.ShapeDtypeStruct((B, 1, NC), jnp.int32),
        ],
        scratch_shapes=[
            pltpu.VMEM((POOL[0], D), jnp.float32),
            pltpu.VMEM((POOL[1], D), jnp.float32),
            pltpu.VMEM((POOL[2], D), jnp.float32),
        ],
    )(x3, emb, n1, n2, wq, wk, wv, wo, w1, w3, w2,
      pq0, pq1, pq2, pwq, pwk, pwv, pwo, cb)


def _dec_kern(h_ref, n1_ref, n2_ref, wq_ref, wk_ref, wv_ref, wo_ref,
              w1_ref, w3_ref, w2_ref, fin_ref, head_ref, o_ref):
    for j in range(DG):
        h = h_ref[j]
        for i in range(L):
            h = _block_body(h, None, n1_ref[i][None], wq_ref[i], wk_ref[i],
                            wv_ref[i], wo_ref[i], n2_ref[i][None],
                            w1_ref[i], w3_ref[i], w2_ref[i])
        xn = _rms(h, fin_ref[...])
        lg = _dot(xn, head_ref[...])
        o_ref[j] = jnp.broadcast_to(lg[:, None, :],
                                    (NC, REP, V)).reshape(S, V)


def _decoder(h, n1, n2, wq, wk, wv, wo, w1, w3, w2, fin_n, head):
    wspec = pl.BlockSpec((L, D, D), lambda b: (0, 0, 0))
    return pl.pallas_call(
        _dec_kern,
        grid=(B // DG,),
        in_specs=[
            pl.BlockSpec((DG, NC, D), lambda b: (b, 0, 0)),
            pl.BlockSpec((L, D), lambda b: (0, 0)),
            pl.BlockSpec((L, D), lambda b: (0, 0)),
            wspec, wspec, wspec, wspec,
            pl.BlockSpec((L, D, F), lambda b: (0, 0, 0)),
            pl.BlockSpec((L, D, F), lambda b: (0, 0, 0)),
            pl.BlockSpec((L, F, D), lambda b: (0, 0, 0)),
            pl.BlockSpec((1, D), lambda b: (0, 0)),
            pl.BlockSpec((D, V), lambda b: (0, 0)),
        ],
        out_specs=pl.BlockSpec((DG, S, V), lambda b: (b, 0, 0)),
        out_shape=jax.ShapeDtypeStruct((B, S, V), jnp.float32),
    )(h, n1, n2, wq, wk, wv, wo, w1, w3, w2, fin_n, head)


def kernel(x, tok_emb, enc_wq, enc_wk, enc_wv, enc_wo, enc_w1, enc_w3,
           enc_w2, enc_n1, enc_n2, dec_wq, dec_wk, dec_wv, dec_wo, dec_w1,
           dec_w3, dec_w2, dec_n1, dec_n2, pq0, pq1, pq2, p_wq, p_wk, p_wv,
           p_wo, codebook, fin_n, head):
    zq, idx3 = _encoder(x.reshape(B, 1, S).astype(jnp.int32), tok_emb,
                        enc_n1, enc_n2, enc_wq, enc_wk, enc_wv, enc_wo,
                        enc_w1, enc_w3, enc_w2, pq0, pq1, pq2, p_wq, p_wk,
                        p_wv, p_wo, codebook)
    logits = _decoder(zq, dec_n1, dec_n2, dec_wq, dec_wk, dec_wv, dec_wo,
                      dec_w1, dec_w3, dec_w2, fin_n[None], head)
    return logits, idx3.reshape(B, NC)


# speedup vs baseline: 1.0274x; 1.0164x over previous
import jax
import jax.numpy as jnp
from jax.experimental import pallas as pl
from jax.experimental.pallas import tpu as pltpu

B = 16
V = 16
D = 512
H = 8
HD = D // H
L = 2
F = 1376
S = 512
K = 512
POOL = (512, 256, 128)
NC = 128
REP = S // NC
EG = 2
DG = 4

_DN_T = (((1,), (1,)), ((), ()))


def _q(x):
    return x.astype(jnp.bfloat16).astype(jnp.float32)


def _dot(a, b):
    return jnp.dot(a, b, preferred_element_type=jnp.float32)


def _dot_t(a, b):
    return jax.lax.dot_general(a, b, _DN_T,
                               preferred_element_type=jnp.float32)


def _split3(x):
    p1 = _q(x)
    r = x - p1
    p2 = _q(r)
    p3 = r - p2
    return p1, p2, p3


def _exact_dot(a_pieces, b):
    p1, p2, p3 = a_pieces
    return (_dot(p1, b) + _dot(p2, b)) + _dot(p3, b)


def _cnt_av(a, cnt, v):
    return _exact_dot(_split3(_q(a) * cnt), _q(v))


def _rms(x, s):
    return (x * jax.lax.rsqrt(jnp.mean(x * x, axis=-1, keepdims=True) + 1e-6)) * s


def _ffn(h, n2, w1, w3, w2):
    xn2 = _rms(h, n2)
    g = _dot(xn2, w1)
    u = _dot(xn2, w3)
    return h + _dot(jax.nn.silu(g) * u, w2)


def _block_body(h, cnt, n1, wq, wk, wv, wo, n2, w1, w3, w2):
    xn = _rms(h, n1)
    q = _dot(xn, wq)
    k = _dot(xn, wk)
    v = _dot(xn, wv)
    scale = 1.0 / (HD ** 0.5)
    outs = []
    for hh in range(H):
        sl = slice(hh * HD, (hh + 1) * HD)
        s = _dot_t(q[:, sl], k[:, sl]) * scale
        e = jnp.exp(s)
        if cnt is None:
            a = e * jax.lax.reciprocal(jnp.sum(e, axis=-1, keepdims=True))
            outs.append(_dot(a, v[:, sl]))
        else:
            a = e * jax.lax.reciprocal(
                jnp.sum(e * cnt, axis=-1, keepdims=True))
            outs.append(_cnt_av(a, cnt, v[:, sl]))
    attn = _dot(jnp.concatenate(outs, axis=1), wo)
    return _ffn(h + attn, n2, w1, w3, w2)


def _pool_attn(q0, h, cnt, wk, wv, wo):
    kt = _dot(h, wk)
    vt = _dot(h, wv)
    s = _dot_t(q0, kt) * (D ** -0.5)
    e = jnp.exp(s)
    if cnt is None:
        a = e * jax.lax.reciprocal(jnp.sum(e, axis=-1, keepdims=True))
        av = _dot(a, vt)
    else:
        a = e * jax.lax.reciprocal(jnp.sum(e * cnt, axis=-1, keepdims=True))
        av = _cnt_av(a, cnt, vt)
    return _dot(av, wo)


def _enc_kern(x_ref, emb_ref, n1_ref, n2_ref, wq_ref, wk_ref, wv_ref,
              wo_ref, w1_ref, w3_ref, w2_ref,
              pq0_ref, pq1_ref, pq2_ref, pwq_ref, pwk_ref, pwv_ref,
              pwo_ref, cb_ref, zq_ref, idx_ref, q0_scr, q1_scr, q2_scr):
    b = pl.program_id(0)

    @pl.when(b == 0)
    def _():
        q0_scr[...] = _dot(pq0_ref[...], pwq_ref[0])
        q1_scr[...] = _dot(pq1_ref[...], pwq_ref[1])
        q2_scr[...] = _dot(pq2_ref[...], pwq_ref[2])

    cb = cb_ref[...]
    cb_pieces = _split3(cb)
    cc = jnp.transpose(jnp.sum(cb * cb, axis=-1, keepdims=True))
    iota = jax.lax.broadcasted_iota(jnp.int32, (V, S), 0)
    for j in range(EG):
        xv = x_ref[j]
        onehot = (iota == xv).astype(jnp.float32)
        cnt = jnp.transpose(jnp.sum(onehot, axis=-1, keepdims=True))

        h = emb_ref[...]
        for i in range(L):
            h = _block_body(h, cnt, n1_ref[i][None], wq_ref[i], wk_ref[i],
                            wv_ref[i], wo_ref[i], n2_ref[i][None], w1_ref[i],
                            w3_ref[i], w2_ref[i])

        h = _pool_attn(q0_scr[...], h, cnt, pwk_ref[0], pwv_ref[0],
                       pwo_ref[0])
        h = _pool_attn(q1_scr[...], h, None, pwk_ref[1], pwv_ref[1],
                       pwo_ref[1])
        h = _pool_attn(q2_scr[...], h, None, pwk_ref[2], pwv_ref[2],
                       pwo_ref[2])

        zz = jnp.sum(h * h, axis=-1, keepdims=True)
        zcb = _dot_t(h, cb)
        d2 = zz - 2.0 * zcb + cc
        idx = jnp.argmin(d2, axis=-1)
        sel = (jax.lax.broadcasted_iota(jnp.int32, (NC, K), 1)
               == idx[:, None]).astype(jnp.float32)
        zq_ref[j] = (_dot(sel, cb_pieces[0]) + _dot(sel, cb_pieces[1])
                     ) + _dot(sel, cb_pieces[2])
        idx_ref[j] = idx.reshape(1, NC)


def _encoder(x3, emb, n1, n2, wq, wk, wv, wo, w1, w3, w2,
             pq0, pq1, pq2, pwq, pwk, pwv, pwo, cb):
    wspec = pl.BlockSpec((L, D, D), lambda b: (0, 0, 0))
    pspec = pl.BlockSpec((3, D, D), lambda b: (0, 0, 0))
    return pl.pallas_call(
        _enc_kern,
        grid=(B // EG,),
        in_specs=[
            pl.BlockSpec((EG, 1, S), lambda b: (b, 0, 0)),
            pl.BlockSpec((V, D), lambda b: (0, 0)),
            pl.BlockSpec((L, D), lambda b: (0, 0)),
            pl.BlockSpec((L, D), lambda b: (0, 0)),
            wspec, wspec, wspec, wspec,
            pl.BlockSpec((L, D, F), lambda b: (0, 0, 0)),
            pl.BlockSpec((L, D, F), lambda b: (0, 0, 0)),
            pl.BlockSpec((L, F, D), lambda b: (0, 0, 0)),
            pl.BlockSpec((POOL[0], D), lambda b: (0, 0)),
            pl.BlockSpec((POOL[1], D), lambda b: (0, 0)),
            pl.BlockSpec((POOL[2], D), lambda b: (0, 0)),
            pspec, pspec, pspec, pspec,
            pl.BlockSpec((K, D), lambda b: (0, 0)),
        ],
        out_specs=[
            pl.BlockSpec((EG, NC, D), lambda b: (b, 0, 0)),
            pl.BlockSpec((EG, 1, NC), lambda b: (b, 0, 0)),
        ],
        out_shape=[
            jax.ShapeDtypeStruct((B, NC, D), jnp.float32),
            jax.ShapeDtypeStruct((B, 1, NC), jnp.int32),
        ],
        scratch_shapes=[
            pltpu.VMEM((POOL[0], D), jnp.float32),
            pltpu.VMEM((POOL[1], D), jnp.float32),
            pltpu.VMEM((POOL[2], D), jnp.float32),
        ],
    )(x3, emb, n1, n2, wq, wk, wv, wo, w1, w3, w2,
      pq0, pq1, pq2, pwq, pwk, pwv, pwo, cb)


def _dec_kern(h_ref, n1_ref, n2_ref, wq_ref, wk_ref, wv_ref, wo_ref,
              w1_ref, w3_ref, w2_ref, fin_ref, head_ref, o_ref):
    for j in range(DG):
        h = h_ref[j]
        for i in range(L):
            h = _block_body(h, None, n1_ref[i][None], wq_ref[i], wk_ref[i],
                            wv_ref[i], wo_ref[i], n2_ref[i][None],
                            w1_ref[i], w3_ref[i], w2_ref[i])
        xn = _rms(h, fin_ref[...])
        lg = _dot(xn, head_ref[...])
        o_ref[j] = jnp.broadcast_to(lg[:, None, :],
                                    (NC, REP, V)).reshape(S, V)


def _decoder(h, n1, n2, wq, wk, wv, wo, w1, w3, w2, fin_n, head):
    wspec = pl.BlockSpec((L, D, D), lambda b: (0, 0, 0))
    return pl.pallas_call(
        _dec_kern,
        grid=(B // DG,),
        in_specs=[
            pl.BlockSpec((DG, NC, D), lambda b: (b, 0, 0)),
            pl.BlockSpec((L, D), lambda b: (0, 0)),
            pl.BlockSpec((L, D), lambda b: (0, 0)),
            wspec, wspec, wspec, wspec,
            pl.BlockSpec((L, D, F), lambda b: (0, 0, 0)),
            pl.BlockSpec((L, D, F), lambda b: (0, 0, 0)),
            pl.BlockSpec((L, F, D), lambda b: (0, 0, 0)),
            pl.BlockSpec((1, D), lambda b: (0, 0)),
            pl.BlockSpec((D, V), lambda b: (0, 0)),
        ],
        out_specs=pl.BlockSpec((DG, S, V), lambda b: (b, 0, 0)),
        out_shape=jax.ShapeDtypeStruct((B, S, V), jnp.float32),
    )(h, n1, n2, wq, wk, wv, wo, w1, w3, w2, fin_n, head)


def kernel(x, tok_emb, enc_wq, enc_wk, enc_wv, enc_wo, enc_w1, enc_w3,
           enc_w2, enc_n1, enc_n2, dec_wq, dec_wk, dec_wv, dec_wo, dec_w1,
           dec_w3, dec_w2, dec_n1, dec_n2, pq0, pq1, pq2, p_wq, p_wk, p_wv,
           p_wo, codebook, fin_n, head):
    zq, idx3 = _encoder(x.reshape(B, 1, S).astype(jnp.int32), tok_emb,
                        enc_n1, enc_n2, enc_wq, enc_wk, enc_wv, enc_wo,
                        enc_w1, enc_w3, enc_w2, pq0, pq1, pq2, p_wq, p_wk,
                        p_wv, p_wo, codebook)
    logits = _decoder(zq, dec_n1, dec_n2, dec_wq, dec_wk, dec_wv, dec_wo,
                      dec_w1, dec_w3, dec_w2, fin_n[None], head)
    return logits, idx3.reshape(B, NC)
